# baseline (speedup 1.0000x reference)
import functools

import jax
import jax.numpy as jnp
from jax import lax
from jax.experimental import pallas as pl
from jax.experimental.pallas import tpu as pltpu
from jax.experimental.pallas import tpu_sc as plsc

_NUM_DOMAINS = 16
_PREFIX_LEN = 32
_HIDDEN = 2048
_VOCAB = 32000
_BATCH = 4
_SEQ = 2048

_NC, _NS = 2, 16
_NW = _NC * _NS
_TOK = _BATCH * _SEQ
_TPW = _TOK // _NW
_WPB = _NW // _BATCH
_CHUNK = 8
_NCHUNK = _TPW // _CHUNK
_NBUF = 7
_GAHEAD = 4
_PPW = 8
_PREF_WORKERS = _BATCH * _PREFIX_LEN // _PPW


def _body(ids_hbm, pidx_hbm, mask_hbm, tok_hbm, pref_hbm,
          out_e_hbm, out_m_hbm,
          idx_v, bufs_v, pidx_v, ones_v, mask_v,
          *sems):
  gsem = list(sems[:_NBUF])
  psem = list(sems[_NBUF:])
  c = lax.axis_index("c")
  s = lax.axis_index("s")
  w = c * _NS + s
  b = w // _WPB
  s0 = (w % _WPB) * _TPW

  pltpu.sync_copy(ids_hbm.at[pl.ds(w * _TPW, _TPW)], idx_v)

  m0 = b * (_PREFIX_LEN + _SEQ)
  pltpu.sync_copy(mask_hbm.at[pl.ds(w * _TPW, _TPW)], mask_v)
  pltpu.sync_copy(mask_v, out_m_hbm.at[pl.ds(m0 + _PREFIX_LEN + s0, _TPW)])

  ones_v[pl.ds(0, 16)] = jnp.ones((16,), jnp.int32)
  ones_v[pl.ds(16, 16)] = jnp.ones((16,), jnp.int32)

  @pl.when(w < _BATCH)
  def _():
    pltpu.sync_copy(ones_v,
                    out_m_hbm.at[pl.ds(w * (_PREFIX_LEN + _SEQ), _PREFIX_LEN)])

  @pl.when(w < _PREF_WORKERS)
  def _():
    pltpu.sync_copy(pidx_hbm.at[pl.ds(w * _PPW, _PPW)], pidx_v)
    prow = bufs_v.at[0, pl.ds(0, _PPW)]
    pltpu.async_copy(pref_hbm.at[pidx_v], prow, gsem[0]).wait()
    b2 = w // (_PREFIX_LEN // _PPW)
    pp0 = (w % (_PREFIX_LEN // _PPW)) * _PPW
    pltpu.sync_copy(prow, out_e_hbm.at[b2, pl.ds(pp0, _PPW)])

  gdesc = [None] * _NCHUNK
  pdesc = [None] * _NCHUNK
  pwaited = [False] * _NCHUNK

  def start_gather(j):
    k = j % _NBUF
    gdesc[j] = pltpu.async_copy(
        tok_hbm.at[idx_v.at[pl.ds(j * _CHUNK, _CHUNK)]], bufs_v.at[k],
        gsem[k])

  for j in range(min(_GAHEAD, _NCHUNK)):
    start_gather(j)

  for j in range(_NCHUNK):
    k = j % _NBUF
    nj = j + _GAHEAD
    if nj < _NCHUNK:
      prev = nj - _NBUF
      if prev >= 0:
        pdesc[prev].wait()
        pwaited[prev] = True
      start_gather(nj)
    gdesc[j].wait()
    r0 = _PREFIX_LEN + s0 + j * _CHUNK
    pdesc[j] = pltpu.async_copy(
        bufs_v.at[k], out_e_hbm.at[b, pl.ds(r0, _CHUNK)], psem[k])

  for j in range(_NCHUNK):
    if not pwaited[j]:
      pdesc[j].wait()


@jax.jit
def _sc_embed(ids, pidx, mask, token_table, pref2d):
  mesh = plsc.VectorSubcoreMesh(core_axis_name="c", subcore_axis_name="s")
  fn = functools.partial(
      pl.kernel,
      out_type=(
          jax.ShapeDtypeStruct((_BATCH, _PREFIX_LEN + _SEQ, _HIDDEN),
                               jnp.float32),
          jax.ShapeDtypeStruct((_BATCH * (_PREFIX_LEN + _SEQ),), jnp.int32),
      ),
      mesh=mesh,
      scratch_types=[
          pltpu.VMEM((_TPW,), jnp.int32),
          pltpu.VMEM((_NBUF, _CHUNK, _HIDDEN), jnp.float32),
          pltpu.VMEM((_PPW,), jnp.int32),
          pltpu.VMEM((_PREFIX_LEN,), jnp.int32),
          pltpu.VMEM((_TPW,), jnp.int32),
      ] + [pltpu.SemaphoreType.DMA] * (2 * _NBUF),
  )(_body)
  return fn(ids, pidx, mask, token_table, pref2d)


def kernel(input_ids, attention_mask, domain_ids, token_table, prefix_table):
  mask_dtype = attention_mask.dtype
  ids = input_ids.astype(jnp.int32).reshape(_TOK)
  mask = attention_mask.astype(jnp.int32).reshape(_TOK)
  dom = domain_ids.astype(jnp.int32)
  pidx = (dom[:, None] * _PREFIX_LEN
          + jnp.arange(_PREFIX_LEN, dtype=jnp.int32)[None, :]).reshape(-1)
  pref2d = prefix_table.reshape(_NUM_DOMAINS * _PREFIX_LEN, _HIDDEN)
  out_e, out_m = _sc_embed(ids, pidx, mask, token_table, pref2d)
  out_m = out_m.reshape(_BATCH, _PREFIX_LEN + _SEQ).astype(mask_dtype)
  return out_e, out_m


# speedup vs baseline: 1.0306x; 1.0306x over previous
import functools

import jax
import jax.numpy as jnp
from jax import lax
from jax.experimental import pallas as pl
from jax.experimental.pallas import tpu as pltpu
from jax.experimental.pallas import tpu_sc as plsc

_NUM_DOMAINS = 16
_PREFIX_LEN = 32
_HIDDEN = 2048
_VOCAB = 32000
_BATCH = 4
_SEQ = 2048

_NC, _NS = 2, 16
_NW = _NC * _NS
_TOK = _BATCH * _SEQ
_TPW = _TOK // _NW
_WPB = _NW // _BATCH
_CHUNK = 16
_NCHUNK = _TPW // _CHUNK
_NBUF = 2
_PPW = 8
_PREF_WORKERS = _BATCH * _PREFIX_LEN // _PPW


def _make_body():
  def body(ids_hbm, mask_hbm, pidx_hbm, tok_hbm, pref_hbm,
           out_e_hbm, out_m_hbm,
           ids_v, idx_f, bufs_v, pbuf_v, pidx_v, min_v, mout_v,
           *sems):
    gsem = list(sems[:_NBUF])
    psem = list(sems[_NBUF:])
    c = lax.axis_index("c")
    s = lax.axis_index("s")
    w = c * _NS + s
    b = w // _WPB
    s0 = (w % _WPB) * _TPW

    pltpu.sync_copy(ids_hbm, ids_v)
    for cc in range(0, _TPW, 16):
      idx_f[pl.ds(cc, 16)] = ids_v[b, pl.ds(s0 + cc, 16)]

    @pl.when(w == _NW - 1)
    def _():
      pltpu.sync_copy(mask_hbm, min_v)
      ones = jnp.ones((16,), jnp.int32)
      for bb in range(_BATCH):
        mout_v[bb, pl.ds(0, 16)] = ones
        mout_v[bb, pl.ds(16, 16)] = ones
        for cc in range(0, _SEQ, 16):
          mout_v[bb, pl.ds(_PREFIX_LEN + cc, 16)] = min_v[bb, pl.ds(cc, 16)]
      pltpu.sync_copy(mout_v, out_m_hbm)

    @pl.when(w < _PREF_WORKERS)
    def _():
      b2 = w // (_PREFIX_LEN // _PPW)
      pp0 = (w % (_PREFIX_LEN // _PPW)) * _PPW
      pltpu.sync_copy(pidx_hbm.at[pl.ds(w * _PPW, _PPW)], pidx_v)
      pltpu.sync_copy(pref_hbm.at[pidx_v], pbuf_v)
      pltpu.sync_copy(pbuf_v, out_e_hbm.at[b2, pl.ds(pp0, _PPW)])

    gdesc = [None] * _NCHUNK
    pdesc = [None] * _NCHUNK
    pwaited = [False] * _NCHUNK

    def start_gather(j):
      k = j % _NBUF
      gdesc[j] = pltpu.async_copy(
          tok_hbm.at[idx_f.at[pl.ds(j * _CHUNK, _CHUNK)]],
          bufs_v.at[k], gsem[k])

    start_gather(0)
    for j in range(_NCHUNK):
      k = j % _NBUF
      nj = j + 1
      if nj < _NCHUNK:
        prev = nj - _NBUF
        if prev >= 0:
          pdesc[prev].wait()
          pwaited[prev] = True
        start_gather(nj)
      gdesc[j].wait()
      r0 = _PREFIX_LEN + s0 + j * _CHUNK
      pdesc[j] = pltpu.async_copy(
          bufs_v.at[k], out_e_hbm.at[b, pl.ds(r0, _CHUNK)], psem[k])

    for j in range(_NCHUNK):
      if not pwaited[j]:
        pdesc[j].wait()

  return body


@jax.jit
def _sc_embed(ids, mask, pidx, token_table, prefix_table):
  mesh = plsc.VectorSubcoreMesh(core_axis_name="c", subcore_axis_name="s")
  fn = functools.partial(
      pl.kernel,
      out_type=(
          jax.ShapeDtypeStruct((_BATCH, _PREFIX_LEN + _SEQ, _HIDDEN),
                               jnp.float32),
          jax.ShapeDtypeStruct((_BATCH, _PREFIX_LEN + _SEQ), jnp.int32),
      ),
      mesh=mesh,
      scratch_types=[
          pltpu.VMEM((_BATCH, _SEQ), jnp.int32),
          pltpu.VMEM((_TPW,), jnp.int32),
          pltpu.VMEM((_NBUF, _CHUNK, _HIDDEN), jnp.float32),
          pltpu.VMEM((_PPW, _HIDDEN), jnp.float32),
          pltpu.VMEM((_PPW,), jnp.int32),
          pltpu.VMEM((_BATCH, _SEQ), jnp.int32),
          pltpu.VMEM((_BATCH, _PREFIX_LEN + _SEQ), jnp.int32),
      ] + [pltpu.SemaphoreType.DMA] * (2 * _NBUF),
  )(_make_body())
  return fn(ids, mask, pidx, token_table, prefix_table)


def kernel(input_ids, attention_mask, domain_ids, token_table, prefix_table):
  mask_dtype = attention_mask.dtype
  pref2d = prefix_table.reshape(_NUM_DOMAINS * _PREFIX_LEN, _HIDDEN)
  dom = domain_ids.astype(jnp.int32)
  pidx = (dom[:, None] * _PREFIX_LEN
          + jnp.arange(_PREFIX_LEN, dtype=jnp.int32)[None, :]).reshape(-1)
  out_e, out_m = _sc_embed(
      input_ids.astype(jnp.int32), attention_mask.astype(jnp.int32),
      pidx, token_table, pref2d)
  return out_e, out_m.astype(mask_dtype)


# speedup vs baseline: 1.0371x; 1.0063x over previous
import functools

import jax
import jax.numpy as jnp
from jax import lax
from jax.experimental import pallas as pl
from jax.experimental.pallas import tpu as pltpu
from jax.experimental.pallas import tpu_sc as plsc

_NUM_DOMAINS = 16
_PREFIX_LEN = 32
_HIDDEN = 2048
_VOCAB = 32000
_BATCH = 4
_SEQ = 2048

_NC, _NS = 2, 16
_NW = _NC * _NS
_TOK = _BATCH * _SEQ
_TPW = _TOK // _NW
_WPB = _NW // _BATCH
_CHUNK = 16
_NCHUNK = _TPW // _CHUNK
_NBUF = 2
_PPW = 8
_PREF_WORKERS = _BATCH * _PREFIX_LEN // _PPW


def _make_body():
  def body(ids_hbm, mask_hbm, pidx_hbm, tok_hbm, pref_hbm,
           out_e_hbm, out_m_hbm,
           ids_v, idx_f, bufs_v, pbuf_v, pidx_v, min_v, mout_v,
           *sems):
    gsem = list(sems[:_NBUF])
    psem = list(sems[_NBUF:])
    c = lax.axis_index("c")
    s = lax.axis_index("s")
    w = c * _NS + s
    b = w // _WPB
    s0 = (w % _WPB) * _TPW

    pltpu.sync_copy(ids_hbm, ids_v)
    def _relocate(i, _):
      cc = i * 16
      idx_f[pl.ds(cc, 16)] = ids_v[b, pl.ds(s0 + cc, 16)]
      return 0
    lax.fori_loop(0, _TPW // 16, _relocate, 0)

    @pl.when(w == _NW - 1)
    def _():
      pltpu.sync_copy(mask_hbm, min_v)
      ones = jnp.ones((16,), jnp.int32)
      for bb in range(_BATCH):
        mout_v[bb, pl.ds(0, 16)] = ones
        mout_v[bb, pl.ds(16, 16)] = ones
      def _mcopy(i, _):
        bb = i // (_SEQ // 16)
        cc = (i % (_SEQ // 16)) * 16
        mout_v[bb, pl.ds(_PREFIX_LEN + cc, 16)] = min_v[bb, pl.ds(cc, 16)]
        return 0
      lax.fori_loop(0, _BATCH * (_SEQ // 16), _mcopy, 0)
      pltpu.sync_copy(mout_v, out_m_hbm)

    @pl.when(w < _PREF_WORKERS)
    def _():
      b2 = w // (_PREFIX_LEN // _PPW)
      pp0 = (w % (_PREFIX_LEN // _PPW)) * _PPW
      pltpu.sync_copy(pidx_hbm.at[pl.ds(w * _PPW, _PPW)], pidx_v)
      pltpu.sync_copy(pref_hbm.at[pidx_v], pbuf_v)
      pltpu.sync_copy(pbuf_v, out_e_hbm.at[b2, pl.ds(pp0, _PPW)])

    gdesc = [None] * _NCHUNK
    pdesc = [None] * _NCHUNK
    pwaited = [False] * _NCHUNK

    def start_gather(j):
      k = j % _NBUF
      gdesc[j] = pltpu.async_copy(
          tok_hbm.at[idx_f.at[pl.ds(j * _CHUNK, _CHUNK)]],
          bufs_v.at[k], gsem[k])

    start_gather(0)
    for j in range(_NCHUNK):
      k = j % _NBUF
      nj = j + 1
      if nj < _NCHUNK:
        prev = nj - _NBUF
        if prev >= 0:
          pdesc[prev].wait()
          pwaited[prev] = True
        start_gather(nj)
      gdesc[j].wait()
      r0 = _PREFIX_LEN + s0 + j * _CHUNK
      pdesc[j] = pltpu.async_copy(
          bufs_v.at[k], out_e_hbm.at[b, pl.ds(r0, _CHUNK)], psem[k])

    for j in range(_NCHUNK):
      if not pwaited[j]:
        pdesc[j].wait()

  return body


@jax.jit
def _sc_embed(ids, mask, pidx, token_table, prefix_table):
  mesh = plsc.VectorSubcoreMesh(core_axis_name="c", subcore_axis_name="s")
  fn = functools.partial(
      pl.kernel,
      out_type=(
          jax.ShapeDtypeStruct((_BATCH, _PREFIX_LEN + _SEQ, _HIDDEN),
                               jnp.float32),
          jax.ShapeDtypeStruct((_BATCH, _PREFIX_LEN + _SEQ), jnp.int32),
      ),
      mesh=mesh,
      scratch_types=[
          pltpu.VMEM((_BATCH, _SEQ), jnp.int32),
          pltpu.VMEM((_TPW,), jnp.int32),
          pltpu.VMEM((_NBUF, _CHUNK, _HIDDEN), jnp.float32),
          pltpu.VMEM((_PPW, _HIDDEN), jnp.float32),
          pltpu.VMEM((_PPW,), jnp.int32),
          pltpu.VMEM((_BATCH, _SEQ), jnp.int32),
          pltpu.VMEM((_BATCH, _PREFIX_LEN + _SEQ), jnp.int32),
      ] + [pltpu.SemaphoreType.DMA] * (2 * _NBUF),
  )(_make_body())
  return fn(ids, mask, pidx, token_table, prefix_table)


def kernel(input_ids, attention_mask, domain_ids, token_table, prefix_table):
  mask_dtype = attention_mask.dtype
  pref2d = prefix_table.reshape(_NUM_DOMAINS * _PREFIX_LEN, _HIDDEN)
  dom = domain_ids.astype(jnp.int32)
  pidx = (dom[:, None] * _PREFIX_LEN
          + jnp.arange(_PREFIX_LEN, dtype=jnp.int32)[None, :]).reshape(-1)
  out_e, out_m = _sc_embed(
      input_ids.astype(jnp.int32), attention_mask.astype(jnp.int32),
      pidx, token_table, pref2d)
  return out_e, out_m.astype(mask_dtype)


# speedup vs baseline: 1.0531x; 1.0155x over previous
import functools

import jax
import jax.numpy as jnp
from jax import lax
from jax.experimental import pallas as pl
from jax.experimental.pallas import tpu as pltpu
from jax.experimental.pallas import tpu_sc as plsc

_NUM_DOMAINS = 16
_PREFIX_LEN = 32
_HIDDEN = 2048
_VOCAB = 32000
_BATCH = 4
_SEQ = 2048

_NC, _NS = 2, 16
_NW = _NC * _NS
_TOK = _BATCH * _SEQ
_TPW = _TOK // _NW
_WPB = _NW // _BATCH
_CHUNK = 8
_NCHUNK = _TPW // _CHUNK
_NBUF = 4
_PPW = 8
_PREF_WORKERS = _BATCH * _PREFIX_LEN // _PPW


def _make_body():
  def body(ids_hbm, mask_hbm, pidx_hbm, tok_hbm, pref_hbm,
           out_e_hbm, out_m_hbm,
           ids_v, idx_f, bufs_v, pbuf_v, pidx_v, min_v, mout_v,
           *sems):
    gsem = list(sems[:_NBUF])
    psem = list(sems[_NBUF:])
    c = lax.axis_index("c")
    s = lax.axis_index("s")
    w = c * _NS + s
    b = w // _WPB
    s0 = (w % _WPB) * _TPW

    pltpu.sync_copy(ids_hbm, ids_v)
    def _relocate(i, _):
      cc = i * 16
      idx_f[pl.ds(cc, 16)] = ids_v[b, pl.ds(s0 + cc, 16)]
      return 0
    lax.fori_loop(0, _TPW // 16, _relocate, 0)

    @pl.when(w == _NW - 1)
    def _():
      pltpu.sync_copy(mask_hbm, min_v)
      ones = jnp.ones((16,), jnp.int32)
      for bb in range(_BATCH):
        mout_v[bb, pl.ds(0, 16)] = ones
        mout_v[bb, pl.ds(16, 16)] = ones
      def _mcopy(i, _):
        bb = i // (_SEQ // 16)
        cc = (i % (_SEQ // 16)) * 16
        mout_v[bb, pl.ds(_PREFIX_LEN + cc, 16)] = min_v[bb, pl.ds(cc, 16)]
        return 0
      lax.fori_loop(0, _BATCH * (_SEQ // 16), _mcopy, 0)
      pltpu.sync_copy(mout_v, out_m_hbm)

    @pl.when(w < _PREF_WORKERS)
    def _():
      b2 = w // (_PREFIX_LEN // _PPW)
      pp0 = (w % (_PREFIX_LEN // _PPW)) * _PPW
      pltpu.sync_copy(pidx_hbm.at[pl.ds(w * _PPW, _PPW)], pidx_v)
      pltpu.sync_copy(pref_hbm.at[pidx_v], pbuf_v)
      pltpu.sync_copy(pbuf_v, out_e_hbm.at[b2, pl.ds(pp0, _PPW)])

    def start_gather(j, k):
      return pltpu.async_copy(
          tok_hbm.at[idx_f.at[pl.ds(j * _CHUNK, _CHUNK)]],
          bufs_v.at[k], gsem[k])

    def put_slice(j):
      return out_e_hbm.at[b, pl.ds(_PREFIX_LEN + s0 + j * _CHUNK, _CHUNK)]

    for t in range(_NBUF - 1):
      start_gather(t, t)

    def ring_body(j2, _):
      a = j2 * _NBUF
      for t in range(_NBUF):
        j = a + t
        nj = j + (_NBUF - 1)
        knj = (t + _NBUF - 1) % _NBUF

        @pl.when(nj < _NCHUNK)
        def _():
          @pl.when(nj >= _NBUF)
          def _():
            pltpu.make_async_copy(bufs_v.at[knj], put_slice(nj - _NBUF),
                                  psem[knj]).wait()
          start_gather(nj, knj)

        pltpu.make_async_copy(tok_hbm.at[idx_f.at[pl.ds(j * _CHUNK, _CHUNK)]],
                              bufs_v.at[t], gsem[t]).wait()
        pltpu.async_copy(bufs_v.at[t], put_slice(j), psem[t])
      return 0

    lax.fori_loop(0, _NCHUNK // _NBUF, ring_body, 0)

    for t in range(_NBUF):
      pltpu.make_async_copy(bufs_v.at[t], put_slice(_NCHUNK - _NBUF + t),
                            psem[t]).wait()

  return body


@jax.jit
def _sc_embed(ids, mask, pidx, token_table, prefix_table):
  mesh = plsc.VectorSubcoreMesh(core_axis_name="c", subcore_axis_name="s")
  fn = functools.partial(
      pl.kernel,
      out_type=(
          jax.ShapeDtypeStruct((_BATCH, _PREFIX_LEN + _SEQ, _HIDDEN),
                               jnp.float32),
          jax.ShapeDtypeStruct((_BATCH, _PREFIX_LEN + _SEQ), jnp.int32),
      ),
      mesh=mesh,
      scratch_types=[
          pltpu.VMEM((_BATCH, _SEQ), jnp.int32),
          pltpu.VMEM((_TPW,), jnp.int32),
          pltpu.VMEM((_NBUF, _CHUNK, _HIDDEN), jnp.float32),
          pltpu.VMEM((_PPW, _HIDDEN), jnp.float32),
          pltpu.VMEM((_PPW,), jnp.int32),
          pltpu.VMEM((_BATCH, _SEQ), jnp.int32),
          pltpu.VMEM((_BATCH, _PREFIX_LEN + _SEQ), jnp.int32),
      ] + [pltpu.SemaphoreType.DMA] * (2 * _NBUF),
  )(_make_body())
  return fn(ids, mask, pidx, token_table, prefix_table)


def kernel(input_ids, attention_mask, domain_ids, token_table, prefix_table):
  mask_dtype = attention_mask.dtype
  pref2d = prefix_table.reshape(_NUM_DOMAINS * _PREFIX_LEN, _HIDDEN)
  dom = domain_ids.astype(jnp.int32)
  pidx = (dom[:, None] * _PREFIX_LEN
          + jnp.arange(_PREFIX_LEN, dtype=jnp.int32)[None, :]).reshape(-1)
  out_e, out_m = _sc_embed(
      input_ids.astype(jnp.int32), attention_mask.astype(jnp.int32),
      pidx, token_table, pref2d)
  return out_e, out_m.astype(mask_dtype)


# speedup vs baseline: 1.0851x; 1.0304x over previous
import functools

import jax
import jax.numpy as jnp
from jax import lax
from jax.experimental import pallas as pl
from jax.experimental.pallas import tpu as pltpu
from jax.experimental.pallas import tpu_sc as plsc

_NUM_DOMAINS = 16
_PREFIX_LEN = 32
_HIDDEN = 2048
_VOCAB = 32000
_BATCH = 4
_SEQ = 2048

_NC, _NS = 2, 16
_NW = _NC * _NS
_TOK = _BATCH * _SEQ
_TPW = _TOK // _NW
_WPB = _NW // _BATCH
_CHUNK = 8
_NCHUNK = _TPW // _CHUNK
_NBUF = 4
_PPW = 8
_PREF_WORKERS = _BATCH * _PREFIX_LEN // _PPW


def _make_body():
  def body(ids_hbm, mask_hbm, dom_hbm, tok_hbm, pref_hbm,
           out_e_hbm, out_m_hbm,
           ids_v, idx_f, bufs_v, pbuf_v, pstage_v, min_v, mout_v, dom16_v,
           *sems):
    gsem = list(sems[:_NBUF])
    psem = list(sems[_NBUF:])
    c = lax.axis_index("c")
    s = lax.axis_index("s")
    w = c * _NS + s
    b = w // _WPB
    s0 = (w % _WPB) * _TPW

    pltpu.sync_copy(ids_hbm, ids_v)
    def _relocate(i, _):
      cc = i * 16
      idx_f[pl.ds(cc, 16)] = ids_v[b, pl.ds(s0 + cc, 16)]
      return 0
    lax.fori_loop(0, _TPW // 16, _relocate, 0)

    @pl.when(w == _NW - 1)
    def _():
      pltpu.sync_copy(mask_hbm, min_v)
      ones = jnp.ones((16,), jnp.int32)
      for bb in range(_BATCH):
        mout_v[bb, pl.ds(0, 16)] = ones
        mout_v[bb, pl.ds(16, 16)] = ones
      def _mcopy(i, _):
        bb = i // (_SEQ // 16)
        cc = (i % (_SEQ // 16)) * 16
        mout_v[bb, pl.ds(_PREFIX_LEN + cc, 16)] = min_v[bb, pl.ds(cc, 16)]
        return 0
      lax.fori_loop(0, _BATCH * (_SEQ // 16), _mcopy, 0)
      pltpu.sync_copy(mout_v, out_m_hbm)

    @pl.when(w < _PREF_WORKERS)
    def _():
      b2 = w // (_PREFIX_LEN // _PPW)
      pp0 = (w % (_PREFIX_LEN // _PPW)) * _PPW
      pltpu.sync_copy(dom_hbm, dom16_v.at[pl.ds(0, _BATCH)])
      dvec = dom16_v[pl.ds(b2, 16)]
      dval = dvec[0]
      pltpu.sync_copy(
          pref_hbm.at[dval, pl.ds(pp0 * _HIDDEN, _PPW * _HIDDEN)], pstage_v)
      def _prelocate(i, _):
        rr = i // (_HIDDEN // 16)
        cc = (i % (_HIDDEN // 16)) * 16
        pbuf_v[rr, pl.ds(cc, 16)] = pstage_v[pl.ds(rr * _HIDDEN + cc, 16)]
        return 0
      lax.fori_loop(0, _PPW * (_HIDDEN // 16), _prelocate, 0)
      pltpu.sync_copy(pbuf_v, out_e_hbm.at[b2, pl.ds(pp0, _PPW)])

    def start_gather(j, k):
      return pltpu.async_copy(
          tok_hbm.at[idx_f.at[pl.ds(j * _CHUNK, _CHUNK)]],
          bufs_v.at[k], gsem[k])

    def put_slice(j):
      return out_e_hbm.at[b, pl.ds(_PREFIX_LEN + s0 + j * _CHUNK, _CHUNK)]

    for t in range(_NBUF - 1):
      start_gather(t, t)

    def ring_body(j2, _):
      a = j2 * _NBUF
      for t in range(_NBUF):
        j = a + t
        nj = j + (_NBUF - 1)
        knj = (t + _NBUF - 1) % _NBUF

        @pl.when(nj < _NCHUNK)
        def _():
          @pl.when(nj >= _NBUF)
          def _():
            pltpu.make_async_copy(bufs_v.at[knj], put_slice(nj - _NBUF),
                                  psem[knj]).wait()
          start_gather(nj, knj)

        pltpu.make_async_copy(tok_hbm.at[idx_f.at[pl.ds(j * _CHUNK, _CHUNK)]],
                              bufs_v.at[t], gsem[t]).wait()
        pltpu.async_copy(bufs_v.at[t], put_slice(j), psem[t])
      return 0

    lax.fori_loop(0, _NCHUNK // _NBUF, ring_body, 0)

    for t in range(_NBUF):
      pltpu.make_async_copy(bufs_v.at[t], put_slice(_NCHUNK - _NBUF + t),
                            psem[t]).wait()

  return body


@jax.jit
def _sc_embed(ids, mask, dom, token_table, prefix_table):
  mesh = plsc.VectorSubcoreMesh(core_axis_name="c", subcore_axis_name="s")
  fn = functools.partial(
      pl.kernel,
      out_type=(
          jax.ShapeDtypeStruct((_BATCH, _PREFIX_LEN + _SEQ, _HIDDEN),
                               jnp.float32),
          jax.ShapeDtypeStruct((_BATCH, _PREFIX_LEN + _SEQ), jnp.int32),
      ),
      mesh=mesh,
      scratch_types=[
          pltpu.VMEM((_BATCH, _SEQ), jnp.int32),
          pltpu.VMEM((_TPW,), jnp.int32),
          pltpu.VMEM((_NBUF, _CHUNK, _HIDDEN), jnp.float32),
          pltpu.VMEM((_PPW, _HIDDEN), jnp.float32),
          pltpu.VMEM((_PPW * _HIDDEN,), jnp.float32),
          pltpu.VMEM((_BATCH, _SEQ), jnp.int32),
          pltpu.VMEM((_BATCH, _PREFIX_LEN + _SEQ), jnp.int32),
          pltpu.VMEM((32,), jnp.int32),
      ] + [pltpu.SemaphoreType.DMA] * (2 * _NBUF),
  )(_make_body())
  return fn(ids, mask, dom, token_table, prefix_table)


def kernel(input_ids, attention_mask, domain_ids, token_table, prefix_table):
  mask_dtype = attention_mask.dtype
  out_e, out_m = _sc_embed(
      input_ids.astype(jnp.int32), attention_mask.astype(jnp.int32),
      domain_ids.astype(jnp.int32), token_table, prefix_table)
  return out_e, out_m.astype(mask_dtype)


# speedup vs baseline: 1.0964x; 1.0104x over previous
import functools

import jax
import jax.numpy as jnp
from jax import lax
from jax.experimental import pallas as pl
from jax.experimental.pallas import tpu as pltpu
from jax.experimental.pallas import tpu_sc as plsc

_NUM_DOMAINS = 16
_PREFIX_LEN = 32
_HIDDEN = 2048
_VOCAB = 32000
_BATCH = 4
_SEQ = 2048

_NC, _NS = 2, 16
_NW = _NC * _NS
_TOK = _BATCH * _SEQ
_TPW = _TOK // _NW
_WPB = _NW // _BATCH
_CHUNK = 8
_NCHUNK = _TPW // _CHUNK
_NBUF = 4
_PPW = 8
_PREF_WORKERS = _BATCH * _PREFIX_LEN // _PPW


def _make_body():
  def body(ids_hbm, mask_hbm, dom_hbm, tok_hbm, pref_hbm,
           out_e_hbm, out_m_hbm,
           idx_f, bufs_v, pbuf_v, pstage_v, min_v, mout_v, dom16_v,
           *sems):
    gsem = list(sems[:_NBUF])
    psem = list(sems[_NBUF:])
    c = lax.axis_index("c")
    s = lax.axis_index("s")
    w = c * _NS + s
    b = w // _WPB
    s0 = (w % _WPB) * _TPW

    pltpu.sync_copy(ids_hbm.at[b, pl.ds(s0, _TPW)], idx_f)

    @pl.when(w == _NW - 1)
    def _():
      pltpu.sync_copy(mask_hbm, min_v)
      ones = jnp.ones((16,), jnp.int32)
      for bb in range(_BATCH):
        mout_v[bb, pl.ds(0, 16)] = ones
        mout_v[bb, pl.ds(16, 16)] = ones
      def _mcopy(i, _):
        bb = i // (_SEQ // 16)
        cc = (i % (_SEQ // 16)) * 16
        mout_v[bb, pl.ds(_PREFIX_LEN + cc, 16)] = min_v[bb, pl.ds(cc, 16)]
        return 0
      lax.fori_loop(0, _BATCH * (_SEQ // 16), _mcopy, 0)
      pltpu.sync_copy(mout_v, out_m_hbm)

    @pl.when(s < _PREF_WORKERS // _NC)
    def _():
      pw = c * (_PREF_WORKERS // _NC) + s
      b2 = pw // (_PREFIX_LEN // _PPW)
      pp0 = (pw % (_PREFIX_LEN // _PPW)) * _PPW
      pltpu.sync_copy(dom_hbm, dom16_v.at[pl.ds(0, _BATCH)])
      dvec = dom16_v[pl.ds(b2, 16)]
      dval = dvec[0]
      pltpu.sync_copy(
          pref_hbm.at[dval, pl.ds(pp0 * _HIDDEN, _PPW * _HIDDEN)], pstage_v)
      def _prelocate(i, _):
        rr = i // (_HIDDEN // 16)
        cc = (i % (_HIDDEN // 16)) * 16
        pbuf_v[rr, pl.ds(cc, 16)] = pstage_v[pl.ds(rr * _HIDDEN + cc, 16)]
        return 0
      lax.fori_loop(0, _PPW * (_HIDDEN // 16), _prelocate, 0)
      pltpu.sync_copy(pbuf_v, out_e_hbm.at[b2, pl.ds(pp0, _PPW)])

    def start_gather(j, k):
      return pltpu.async_copy(
          tok_hbm.at[idx_f.at[pl.ds(j * _CHUNK, _CHUNK)]],
          bufs_v.at[k], gsem[k])

    def put_slice(j):
      return out_e_hbm.at[b, pl.ds(_PREFIX_LEN + s0 + j * _CHUNK, _CHUNK)]

    for t in range(_NBUF - 1):
      start_gather(t, t)

    def ring_body(j2, _):
      a = j2 * _NBUF
      for t in range(_NBUF):
        j = a + t
        nj = j + (_NBUF - 1)
        knj = (t + _NBUF - 1) % _NBUF

        @pl.when(nj < _NCHUNK)
        def _():
          @pl.when(nj >= _NBUF)
          def _():
            pltpu.make_async_copy(bufs_v.at[knj], put_slice(nj - _NBUF),
                                  psem[knj]).wait()
          start_gather(nj, knj)

        pltpu.make_async_copy(tok_hbm.at[idx_f.at[pl.ds(j * _CHUNK, _CHUNK)]],
                              bufs_v.at[t], gsem[t]).wait()
        pltpu.async_copy(bufs_v.at[t], put_slice(j), psem[t])
      return 0

    lax.fori_loop(0, _NCHUNK // _NBUF, ring_body, 0)

    for t in range(_NBUF):
      pltpu.make_async_copy(bufs_v.at[t], put_slice(_NCHUNK - _NBUF + t),
                            psem[t]).wait()

  return body


@jax.jit
def _sc_embed(ids, mask, dom, token_table, prefix_table):
  mesh = plsc.VectorSubcoreMesh(core_axis_name="c", subcore_axis_name="s")
  fn = functools.partial(
      pl.kernel,
      out_type=(
          jax.ShapeDtypeStruct((_BATCH, _PREFIX_LEN + _SEQ, _HIDDEN),
                               jnp.float32),
          jax.ShapeDtypeStruct((_BATCH, _PREFIX_LEN + _SEQ), jnp.int32),
      ),
      mesh=mesh,
      scratch_types=[
          pltpu.VMEM((_TPW,), jnp.int32),
          pltpu.VMEM((_NBUF, _CHUNK, _HIDDEN), jnp.float32),
          pltpu.VMEM((_PPW, _HIDDEN), jnp.float32),
          pltpu.VMEM((_PPW * _HIDDEN,), jnp.float32),
          pltpu.VMEM((_BATCH, _SEQ), jnp.int32),
          pltpu.VMEM((_BATCH, _PREFIX_LEN + _SEQ), jnp.int32),
          pltpu.VMEM((32,), jnp.int32),
      ] + [pltpu.SemaphoreType.DMA] * (2 * _NBUF),
  )(_make_body())
  return fn(ids, mask, dom, token_table, prefix_table)


def kernel(input_ids, attention_mask, domain_ids, token_table, prefix_table):
  mask_dtype = attention_mask.dtype
  out_e, out_m = _sc_embed(
      input_ids.astype(jnp.int32), attention_mask.astype(jnp.int32),
      domain_ids.astype(jnp.int32), token_table, prefix_table)
  return out_e, out_m.astype(mask_dtype)
